# baseline (speedup 1.0000x reference)
import jax
import jax.numpy as jnp
from jax import lax
from jax.experimental import pallas as pl
from jax.experimental.pallas import tpu as pltpu

B, D, T, K = 32, 64, 1024, 1024
BB = 4
TT = BB * T


def _vq_body(x_ref, emb_ref, embm2_ref, embaug_ref, idx_ref, val_ref, loss_ref):
    b = pl.program_id(0)
    emb = emb_ref[...]
    e2 = jnp.sum(emb * emb, axis=1)
    part = jnp.float32(0.0)
    cnts = []
    idxs = []
    for h in range(2):
        hb = BB // 2
        xb = jnp.concatenate(
            [x_ref[h * hb + i] for i in range(hb)], axis=1)
        x2 = jnp.sum(xb * xb, axis=0)
        m2 = lax.dot_general(embm2_ref[...], xb, (((1,), (0,)), ((), ())),
                             preferred_element_type=jnp.float32)
        dist = (x2[None, :] + e2[:, None]) + m2
        minv = jnp.min(dist, axis=0)
        onehot = jnp.where(dist == minv[None, :], 1.0, 0.0)

        aug = lax.dot_general(embaug_ref[...], onehot, (((0,), (0,)), ((), ())),
                              preferred_element_type=jnp.float32)
        vals = aug[:D]
        cnt = aug[D]
        s1 = aug[D + 1] * 256.0 + aug[D + 2]
        s2 = (aug[D + 3] * 65536.0 + aug[D + 4] * 256.0) + aug[D + 5]
        k1 = (s1 - jnp.sqrt(cnt * s2 - s1 * s1)) / cnt
        idx = k1.astype(jnp.int32)
        idxs.append(idx)
        cnts.append(cnt)
        for i in range(hb):
            val_ref[h * hb + i] = vals[:, i * T:(i + 1) * T]
        diff = xb - vals
        part = part + jnp.sum(diff * diff)
    idx_ref[...] = jnp.concatenate(idxs).reshape(BB, 1, T)
    idx = jnp.concatenate(idxs)
    cnt = jnp.concatenate(cnts)

    @pl.when(jnp.max(cnt) > 1.5)
    def _():
        kiota = lax.broadcasted_iota(jnp.int32, (K, TT), 0)
        onehot2 = jnp.where(kiota == idx[None, :], 1.0, 0.0)
        vals2 = lax.dot_general(emb, onehot2, (((0,), (0,)), ((), ())),
                                preferred_element_type=jnp.float32)
        for i in range(BB):
            val_ref[i] = vals2[:, i * T:(i + 1) * T]

    @pl.when(b == 0)
    def _():
        loss_ref[0, 0] = part

    @pl.when(b > 0)
    def _():
        loss_ref[0, 0] += part

    @pl.when(b == (B // BB) - 1)
    def _():
        loss_ref[0, 0] = loss_ref[0, 0] * (2.0 / (B * T * D))


@jax.jit
def kernel(x, embedding):
    embm2 = embedding * (-2.0)
    kv = lax.iota(jnp.int32, K)
    ksq = kv * kv
    digits = jnp.stack(
        [jnp.ones((K,), jnp.int32), kv // 256, kv % 256,
         ksq // 65536, (ksq % 65536) // 256, ksq % 256],
        axis=1).astype(jnp.float32)
    embaug = jnp.concatenate([embedding, digits], axis=1)
    idx3, values, loss = pl.pallas_call(
        _vq_body,
        grid=(B // BB,),
        in_specs=[
            pl.BlockSpec((BB, D, T), lambda b: (b, 0, 0)),
            pl.BlockSpec((K, D), lambda b: (0, 0)),
            pl.BlockSpec((K, D), lambda b: (0, 0)),
            pl.BlockSpec((K, D + 6), lambda b: (0, 0)),
        ],
        out_specs=[
            pl.BlockSpec((BB, 1, T), lambda b: (b, 0, 0)),
            pl.BlockSpec((BB, D, T), lambda b: (b, 0, 0)),
            pl.BlockSpec(memory_space=pltpu.SMEM, block_shape=(1, 1),
                         index_map=lambda b: (0, 0)),
        ],
        out_shape=[
            jax.ShapeDtypeStruct((B, 1, T), jnp.int32),
            jax.ShapeDtypeStruct((B, D, T), jnp.float32),
            jax.ShapeDtypeStruct((1, 1), jnp.float32),
        ],
    )(x, embedding, embm2, embaug)
    return values, idx3.reshape(B, T), loss[0, 0]


# speedup vs baseline: 1.0286x; 1.0286x over previous
import jax
import jax.numpy as jnp
from jax import lax
from jax.experimental import pallas as pl
from jax.experimental.pallas import tpu as pltpu

B, D, T, K = 32, 64, 1024, 1024
BB = 4
TT = BB * T


def _vq_body(x_ref, emb_ref, embm2_ref, embaug_ref, idx_ref, val_ref, loss_ref):
    b = pl.program_id(0)
    xb = jnp.concatenate([x_ref[i] for i in range(BB)], axis=1)
    emb = emb_ref[...]
    e2 = jnp.sum(emb * emb, axis=1)
    x2 = jnp.sum(xb * xb, axis=0)
    m2 = lax.dot_general(embm2_ref[...], xb, (((1,), (0,)), ((), ())),
                         preferred_element_type=jnp.float32)
    dist = (x2[None, :] + e2[:, None]) + m2
    minv = jnp.min(dist, axis=0)
    onehot = jnp.where(dist == minv[None, :], 1.0, 0.0)

    aug = lax.dot_general(embaug_ref[...], onehot, (((0,), (0,)), ((), ())),
                          preferred_element_type=jnp.float32)
    vals = aug[:D]
    cnt = aug[D]
    s1 = aug[D + 1] * 256.0 + aug[D + 2]
    s2 = (aug[D + 3] * 65536.0 + aug[D + 4] * 256.0) + aug[D + 5]
    k1 = (s1 - jnp.sqrt(cnt * s2 - s1 * s1)) / cnt
    idx = k1.astype(jnp.int32)
    idx_ref[...] = idx.reshape(BB, 1, T)

    for i in range(BB):
        val_ref[i] = vals[:, i * T:(i + 1) * T]

    diff = xb - vals
    part = jnp.sum(diff * diff)

    @pl.when(jnp.max(cnt) > 1.5)
    def _():
        kiota = lax.broadcasted_iota(jnp.int32, (K, TT), 0)
        onehot2 = jnp.where(kiota == idx[None, :], 1.0, 0.0)
        vals2 = lax.dot_general(emb, onehot2, (((0,), (0,)), ((), ())),
                                preferred_element_type=jnp.float32)
        for i in range(BB):
            val_ref[i] = vals2[:, i * T:(i + 1) * T]

    @pl.when(b == 0)
    def _():
        loss_ref[0, 0] = part

    @pl.when(b > 0)
    def _():
        loss_ref[0, 0] += part

    @pl.when(b == (B // BB) - 1)
    def _():
        loss_ref[0, 0] = loss_ref[0, 0] * (2.0 / (B * T * D))


@jax.jit
def kernel(x, embedding):
    embm2 = embedding * (-2.0)
    kv = lax.iota(jnp.int32, K)
    ksq = kv * kv
    digits = jnp.stack(
        [jnp.ones((K,), jnp.int32), kv // 256, kv % 256,
         ksq // 65536, (ksq % 65536) // 256, ksq % 256],
        axis=1).astype(jnp.float32)
    embaug = jnp.concatenate([embedding, digits], axis=1)
    idx3, values, loss = pl.pallas_call(
        _vq_body,
        grid=(B // BB,),
        in_specs=[
            pl.BlockSpec((BB, D, T), lambda b: (b, 0, 0)),
            pl.BlockSpec((K, D), lambda b: (0, 0)),
            pl.BlockSpec((K, D), lambda b: (0, 0)),
            pl.BlockSpec((K, D + 6), lambda b: (0, 0)),
        ],
        out_specs=[
            pl.BlockSpec((BB, 1, T), lambda b: (b, 0, 0)),
            pl.BlockSpec((BB, D, T), lambda b: (b, 0, 0)),
            pl.BlockSpec(memory_space=pltpu.SMEM, block_shape=(1, 1),
                         index_map=lambda b: (0, 0)),
        ],
        out_shape=[
            jax.ShapeDtypeStruct((B, 1, T), jnp.int32),
            jax.ShapeDtypeStruct((B, D, T), jnp.float32),
            jax.ShapeDtypeStruct((1, 1), jnp.float32),
        ],
    )(x, embedding, embm2, embaug)
    return values, idx3.reshape(B, T), loss[0, 0]
